# baseline (speedup 1.0000x reference)
import functools
import math

import jax
import jax.numpy as jnp
from jax import lax
from jax.experimental import pallas as pl
from jax.experimental.pallas import tpu as pltpu
from jax.experimental.pallas import tpu_sc as plsc

VOCAB = 100000
D = 128
B = 4
T = 2048
NC, NS, L = 2, 16, 16
NW = NC * NS
PW = T // NW
NQ = 8
QW = PW // NQ
SCALE = math.sqrt(D)

_mesh = plsc.VectorSubcoreMesh(core_axis_name="c", subcore_axis_name="s")


@functools.partial(
    pl.kernel,
    mesh=_mesh,
    out_type=jax.ShapeDtypeStruct((B, T, D), jnp.float32),
    scratch_types=[
        pltpu.VMEM((B, PW), jnp.int32),
        pltpu.VMEM((B * PW, D), jnp.float32),
        pltpu.VMEM((PW, D), jnp.float32),
        pltpu.SemaphoreType.DMA,
        pltpu.SemaphoreType.DMA,
        pltpu.SemaphoreType.DMA,
        pltpu.SemaphoreType.DMA,
        pltpu.SemaphoreType.DMA,
        pltpu.SemaphoreType.DMA,
        pltpu.SemaphoreType.DMA,
        pltpu.SemaphoreType.DMA,
        pltpu.SemaphoreType.DMA,
        pltpu.SemaphoreType.DMA,
        pltpu.SemaphoreType.DMA,
    ],
)
def _embed(idx_hbm, tok_hbm, pos_hbm, out_hbm, idx_v, rows_v, pos_v,
           isem, psem, q0, q1, q2, q3, q4, q5, q6, q7, osem):
    wid = lax.axis_index("s") * NC + lax.axis_index("c")
    p0 = wid * PW
    qsems = (q0, q1, q2, q3, q4, q5, q6, q7)

    pcopy = pltpu.async_copy(pos_hbm.at[pl.ds(p0, PW)], pos_v, psem)
    icopies = [
        pltpu.async_copy(idx_hbm.at[b, pl.ds(p0, PW)], idx_v.at[b], isem)
        for b in range(B)
    ]
    for c in icopies:
        c.wait()

    gathers = [
        [
            pltpu.async_copy(
                tok_hbm.at[idx_v.at[b, pl.ds(q * QW, QW)]],
                rows_v.at[pl.ds(b * PW + q * QW, QW)], qsems[q])
            for b in range(B)
        ]
        for q in range(NQ)
    ]

    out_waits = []
    for q in range(NQ):
        for g in gathers[q]:
            g.wait()
        if q == 0:
            pcopy.wait()

        def body(i, carry, q=q):
            pi = q * QW + i
            for j in range(D // L):
                sl = pl.ds(j * L, L)
                pv = pos_v[pi, sl]
                for b in range(B):
                    row = b * PW + pi
                    rows_v[row, sl] = rows_v[row, sl] * SCALE + pv
            return carry

        lax.fori_loop(0, QW, body, 0)
        for b in range(B):
            out_waits.append(pltpu.async_copy(
                rows_v.at[pl.ds(b * PW + q * QW, QW)],
                out_hbm.at[b, pl.ds(p0 + q * QW, QW)], osem))

    for w in out_waits:
        w.wait()


def kernel(token_ids, tok_table, pos_table):
    out = _embed(token_ids.astype(jnp.int32), tok_table, pos_table)
    return out


# speedup vs baseline: 1.0397x; 1.0397x over previous
import functools
import math

import jax
import jax.numpy as jnp
from jax import lax
from jax.experimental import pallas as pl
from jax.experimental.pallas import tpu as pltpu
from jax.experimental.pallas import tpu_sc as plsc

VOCAB = 100000
D = 128
B = 4
T = 2048
NC, NS, L = 2, 16, 16
NW = NC * NS
PW = T // NW
NQ = 2
QW = PW // NQ
SCALE = math.sqrt(D)

_mesh = plsc.VectorSubcoreMesh(core_axis_name="c", subcore_axis_name="s")


@functools.partial(
    pl.kernel,
    mesh=_mesh,
    out_type=jax.ShapeDtypeStruct((B, T, D), jnp.float32),
    scratch_types=[
        pltpu.VMEM((B, PW), jnp.int32),
        pltpu.VMEM((B * PW, D), jnp.float32),
        pltpu.VMEM((PW, D), jnp.float32),
        pltpu.SemaphoreType.DMA,
        pltpu.SemaphoreType.DMA,
        pltpu.SemaphoreType.DMA,
        pltpu.SemaphoreType.DMA,
        pltpu.SemaphoreType.DMA,
    ],
)
def _embed(idx_hbm, tok_hbm, pos_hbm, out_hbm, idx_v, rows_v, pos_v,
           isem, psem, q0, q1, osem):
    wid = lax.axis_index("s") * NC + lax.axis_index("c")
    p0 = wid * PW
    qsems = (q0, q1)

    pcopy = pltpu.async_copy(pos_hbm.at[pl.ds(p0, PW)], pos_v, psem)
    icopies = [
        pltpu.async_copy(idx_hbm.at[b, pl.ds(p0, PW)], idx_v.at[b], isem)
        for b in range(B)
    ]
    for c in icopies:
        c.wait()

    gathers = [
        [
            pltpu.async_copy(
                tok_hbm.at[idx_v.at[b, pl.ds(q * QW, QW)]],
                rows_v.at[pl.ds(b * PW + q * QW, QW)], qsems[q])
            for b in range(B)
        ]
        for q in range(NQ)
    ]

    out_waits = []
    for q in range(NQ):
        for g in gathers[q]:
            g.wait()
        if q == 0:
            pcopy.wait()

        def body(i, carry, q=q):
            pi = q * QW + i
            for j in range(D // L):
                sl = pl.ds(j * L, L)
                pv = pos_v[pi, sl]
                for b in range(B):
                    row = b * PW + pi
                    rows_v[row, sl] = rows_v[row, sl] * SCALE + pv
            return carry

        lax.fori_loop(0, QW, body, 0)
        for b in range(B):
            out_waits.append(pltpu.async_copy(
                rows_v.at[pl.ds(b * PW + q * QW, QW)],
                out_hbm.at[b, pl.ds(p0 + q * QW, QW)], osem))

    for w in out_waits:
        w.wait()


def kernel(token_ids, tok_table, pos_table):
    out = _embed(token_ids.astype(jnp.int32), tok_table, pos_table)
    return out


# speedup vs baseline: 1.0411x; 1.0014x over previous
import functools
import math

import jax
import jax.numpy as jnp
from jax import lax
from jax.experimental import pallas as pl
from jax.experimental.pallas import tpu as pltpu
from jax.experimental.pallas import tpu_sc as plsc

VOCAB = 100000
D = 128
B = 4
T = 2048
NC, NS, L = 2, 16, 16
NW = NC * NS
PW = T // NW
NQ = 1
QW = PW // NQ
SCALE = math.sqrt(D)

_mesh = plsc.VectorSubcoreMesh(core_axis_name="c", subcore_axis_name="s")


@functools.partial(
    pl.kernel,
    mesh=_mesh,
    out_type=jax.ShapeDtypeStruct((B, T, D), jnp.float32),
    scratch_types=[
        pltpu.VMEM((B, PW), jnp.int32),
        pltpu.VMEM((B * PW, D), jnp.float32),
        pltpu.VMEM((PW, D), jnp.float32),
        pltpu.SemaphoreType.DMA,
        pltpu.SemaphoreType.DMA,
        pltpu.SemaphoreType.DMA,
        pltpu.SemaphoreType.DMA,
    ],
)
def _embed(idx_hbm, tok_hbm, pos_hbm, out_hbm, idx_v, rows_v, pos_v,
           isem, psem, q0, osem):
    wid = lax.axis_index("s") * NC + lax.axis_index("c")
    p0 = wid * PW
    qsems = (q0,)

    pcopy = pltpu.async_copy(pos_hbm.at[pl.ds(p0, PW)], pos_v, psem)
    icopies = [
        pltpu.async_copy(idx_hbm.at[b, pl.ds(p0, PW)], idx_v.at[b], isem)
        for b in range(B)
    ]
    for c in icopies:
        c.wait()

    gathers = [
        [
            pltpu.async_copy(
                tok_hbm.at[idx_v.at[b, pl.ds(q * QW, QW)]],
                rows_v.at[pl.ds(b * PW + q * QW, QW)], qsems[q])
            for b in range(B)
        ]
        for q in range(NQ)
    ]

    out_waits = []
    for q in range(NQ):
        for g in gathers[q]:
            g.wait()
        if q == 0:
            pcopy.wait()

        def body(i, carry, q=q):
            pi = q * QW + i
            for j in range(D // L):
                sl = pl.ds(j * L, L)
                pv = pos_v[pi, sl]
                for b in range(B):
                    row = b * PW + pi
                    rows_v[row, sl] = rows_v[row, sl] * SCALE + pv
            return carry

        lax.fori_loop(0, QW, body, 0)
        for b in range(B):
            out_waits.append(pltpu.async_copy(
                rows_v.at[pl.ds(b * PW + q * QW, QW)],
                out_hbm.at[b, pl.ds(p0 + q * QW, QW)], osem))

    for w in out_waits:
        w.wait()


def kernel(token_ids, tok_table, pos_table):
    out = _embed(token_ids.astype(jnp.int32), tok_table, pos_table)
    return out
